# baseline (speedup 1.0000x reference)
import jax
import jax.numpy as jnp
from jax.experimental import pallas as pl
from jax.experimental.pallas import tpu as pltpu

_B, _N = 8, 2048
_BR = 256
_NBLK = _N // _BR
_EPS = 1e-12


def _bf(x):
    return x.astype(jnp.bfloat16)


def _sel_dist(rows_t, cols):
    bb = jnp.sum(rows_t * rows_t, axis=0, keepdims=True)
    aa = jnp.sum(cols * cols, axis=1, keepdims=True)
    ab = jax.lax.dot_general(
        _bf(cols), _bf(rows_t), (((1,), (0,)), ((), ())),
        preferred_element_type=jnp.float32)
    return (aa + bb) - 2.0 * ab


def _argmin_mask(dmat, iota):
    jmin = jnp.argmin(dmat, axis=1).astype(jnp.int32)[:, None]
    return iota == jmin


def _limbs(x):
    hi = x.astype(jnp.bfloat16)
    r1 = x - hi.astype(jnp.float32)
    mid = r1.astype(jnp.bfloat16)
    lo = (r1 - mid.astype(jnp.float32)).astype(jnp.bfloat16)
    return hi, mid, lo


def _pick(mask, tbl):
    e = jax.lax.dot_general(
        _bf(mask), tbl, (((1,), (0,)), ((), ())),
        preferred_element_type=jnp.float32)
    return ((e[:, 0:1] + e[:, 3:4]) + e[:, 6:7],
            (e[:, 1:2] + e[:, 4:5]) + e[:, 7:8],
            (e[:, 2:3] + e[:, 5:6]) + e[:, 8:9])


def _contrib(mask, tbl, cols, nrm):
    px, py, pz = _pick(mask, tbl)
    dx = px - cols[:, 0:1]
    dy = py - cols[:, 1:2]
    dz = pz - cols[:, 2:3]
    norm = jnp.sqrt(dx * dx + dy * dy + dz * dz)
    inv = 1.0 / jnp.maximum(norm, _EPS)
    return jnp.abs((dx * inv) * nrm[:, 0:1] + (dy * inv) * nrm[:, 1:2]
                   + (dz * inv) * nrm[:, 2:3])


def _cloud_kappa(rows_t, tbl, cols, nrm, iota):
    inf = jnp.float32(jnp.inf)
    dsel = _sel_dist(rows_t, cols)
    m1 = _argmin_mask(dsel, iota)
    dm = jnp.where(m1, inf, dsel)
    m2 = _argmin_mask(dm, iota)
    dm2 = jnp.where(m2, inf, dm)
    m3 = _argmin_mask(dm2, iota)
    return (_contrib(m2, tbl, cols, nrm) + _contrib(m3, tbl, cols, nrm)) * 0.5


def _body(ot_ref, at_ref, of_ref, af_ref, nf_ref, o_ref, a_ref, n_ref,
          ko_ref, ka_ref):
    pt = ot_ref[0]
    at = at_ref[0]
    tblo = jnp.concatenate(_limbs(of_ref[0]), axis=1)
    tbla = jnp.concatenate(_limbs(af_ref[0]), axis=1)
    tbln = jnp.concatenate(_limbs(nf_ref[0]), axis=1)
    ob = o_ref[0]
    ab_ = a_ref[0]
    nb = n_ref[0]
    iota = jax.lax.broadcasted_iota(jnp.int32, (_BR, _N), 1)

    ko = _cloud_kappa(pt, tblo, ob, nb, iota)

    dao = _sel_dist(pt, ab_)
    nhx, nhy, nhz = _pick(_argmin_mask(dao, iota), tbln)
    nh = jnp.concatenate([nhx, nhy, nhz], axis=1)

    ka = _cloud_kappa(at, tbla, ab_, nh, iota)

    ko_ref[...] = ko.reshape(1, _BR, 1)
    ka_ref[...] = ka.reshape(1, _BR, 1)


def _std_body(ko_ref, ka_ref, o_ref):
    n = jnp.float32(_N)

    def _std(x):
        mean = jnp.sum(x, axis=1, keepdims=True) / n
        d = x - mean
        return jnp.sqrt(jnp.sum(d * d, axis=1, keepdims=True) / (n - 1.0))

    diff = jnp.abs(_std(ka_ref[...]) - _std(ko_ref[...]))
    o_ref[...] = jnp.sum(diff, axis=0, keepdims=True) / jnp.float32(_B)


def _call(ori_pcs, adv_pcs, ori_normals, interpret=False):
    ori_t = ori_pcs.transpose(0, 2, 1)
    adv_t = adv_pcs.transpose(0, 2, 1)
    ko, ka = pl.pallas_call(
        _body,
        grid=(_B, _NBLK),
        in_specs=[
            pl.BlockSpec((1, 3, _N), lambda b, r: (b, 0, 0)),
            pl.BlockSpec((1, 3, _N), lambda b, r: (b, 0, 0)),
            pl.BlockSpec((1, _N, 3), lambda b, r: (b, 0, 0)),
            pl.BlockSpec((1, _N, 3), lambda b, r: (b, 0, 0)),
            pl.BlockSpec((1, _N, 3), lambda b, r: (b, 0, 0)),
            pl.BlockSpec((1, _BR, 3), lambda b, r: (b, r, 0)),
            pl.BlockSpec((1, _BR, 3), lambda b, r: (b, r, 0)),
            pl.BlockSpec((1, _BR, 3), lambda b, r: (b, r, 0)),
        ],
        out_specs=[pl.BlockSpec((1, _BR, 1), lambda b, r: (b, r, 0)),
                   pl.BlockSpec((1, _BR, 1), lambda b, r: (b, r, 0))],
        out_shape=[jax.ShapeDtypeStruct((_B, _N, 1), jnp.float32),
                   jax.ShapeDtypeStruct((_B, _N, 1), jnp.float32)],
        compiler_params=pltpu.CompilerParams(
            dimension_semantics=("arbitrary", "arbitrary")),
        interpret=interpret,
    )(ori_t, adv_t, ori_pcs, adv_pcs, ori_normals,
      ori_pcs, adv_pcs, ori_normals)
    out = pl.pallas_call(
        _std_body,
        out_shape=jax.ShapeDtypeStruct((1, 1), jnp.float32),
        interpret=interpret,
    )(ko.reshape(_B, _N), ka.reshape(_B, _N))
    return out[0, 0]


def kernel(ori_pcs, adv_pcs, ori_normals):
    return _call(ori_pcs, adv_pcs, ori_normals)


# speedup vs baseline: 1.7598x; 1.7598x over previous
import jax
import jax.numpy as jnp
from jax.experimental import pallas as pl
from jax.experimental.pallas import tpu as pltpu

_B, _N = 8, 2048
_BR = 256
_NBLK = _N // _BR
_EPS = 1e-12


def _bf(x):
    return x.astype(jnp.bfloat16)


def _sel_dist(rows_t, cols):
    bb = jnp.sum(rows_t * rows_t, axis=0, keepdims=True)
    aa = jnp.sum(cols * cols, axis=1, keepdims=True)
    ab = jax.lax.dot_general(
        _bf(cols), _bf(rows_t), (((1,), (0,)), ((), ())),
        preferred_element_type=jnp.float32)
    return (aa + bb) - 2.0 * ab


def _argmin_mask(dmat, iota):
    jmin = jnp.argmin(dmat, axis=1).astype(jnp.int32)[:, None]
    return iota == jmin


def _limbs(x):
    hi = x.astype(jnp.bfloat16)
    r1 = x - hi.astype(jnp.float32)
    mid = r1.astype(jnp.bfloat16)
    lo = (r1 - mid.astype(jnp.float32)).astype(jnp.bfloat16)
    return hi, mid, lo


def _pick(mask, tbl):
    e = jax.lax.dot_general(
        _bf(mask), tbl, (((1,), (0,)), ((), ())),
        preferred_element_type=jnp.float32)
    return ((e[:, 0:1] + e[:, 3:4]) + e[:, 6:7],
            (e[:, 1:2] + e[:, 4:5]) + e[:, 7:8],
            (e[:, 2:3] + e[:, 5:6]) + e[:, 8:9])


def _contrib(mask, tbl, cols, nrm):
    px, py, pz = _pick(mask, tbl)
    dx = px - cols[:, 0:1]
    dy = py - cols[:, 1:2]
    dz = pz - cols[:, 2:3]
    norm = jnp.sqrt(dx * dx + dy * dy + dz * dz)
    inv = 1.0 / jnp.maximum(norm, _EPS)
    return jnp.abs((dx * inv) * nrm[:, 0:1] + (dy * inv) * nrm[:, 1:2]
                   + (dz * inv) * nrm[:, 2:3])


def _cloud_kappa(rows_t, tbl, cols, nrm, iota):
    inf = jnp.float32(jnp.inf)
    dsel = _sel_dist(rows_t, cols)
    m1 = _argmin_mask(dsel, iota)
    dm = jnp.where(m1, inf, dsel)
    m2 = _argmin_mask(dm, iota)
    dm2 = jnp.where(m2, inf, dm)
    m3 = _argmin_mask(dm2, iota)
    return (_contrib(m2, tbl, cols, nrm) + _contrib(m3, tbl, cols, nrm)) * 0.5


def _body(ot_ref, at_ref, of_ref, af_ref, nf_ref, o_ref, a_ref, n_ref,
          ko_ref, ka_ref):
    pt = ot_ref[0]
    at = at_ref[0]
    tblo = of_ref[0]
    tbla = af_ref[0]
    tbln = nf_ref[0]
    ob = o_ref[0]
    ab_ = a_ref[0]
    nb = n_ref[0]
    iota = jax.lax.broadcasted_iota(jnp.int32, (_BR, _N), 1)

    ko = _cloud_kappa(pt, tblo, ob, nb, iota)

    dao = _sel_dist(pt, ab_)
    nhx, nhy, nhz = _pick(_argmin_mask(dao, iota), tbln)
    nh = jnp.concatenate([nhx, nhy, nhz], axis=1)

    ka = _cloud_kappa(at, tbla, ab_, nh, iota)

    ko_ref[...] = ko.reshape(1, _BR, 1)
    ka_ref[...] = ka.reshape(1, _BR, 1)


def _limb_body(o_in, a_in, n_in, o_out, a_out, n_out):
    for x_ref, t_ref in ((o_in, o_out), (a_in, a_out), (n_in, n_out)):
        hi, mid, lo = _limbs(x_ref[0])
        t_ref[...] = jnp.concatenate([hi, mid, lo], axis=1).reshape(1, _N, 9)


def _std_body(ko_ref, ka_ref, o_ref):
    n = jnp.float32(_N)

    def _std(x):
        mean = jnp.sum(x, axis=1, keepdims=True) / n
        d = x - mean
        return jnp.sqrt(jnp.sum(d * d, axis=1, keepdims=True) / (n - 1.0))

    diff = jnp.abs(_std(ka_ref[...]) - _std(ko_ref[...]))
    o_ref[...] = jnp.sum(diff, axis=0, keepdims=True) / jnp.float32(_B)


def _call(ori_pcs, adv_pcs, ori_normals, interpret=False):
    ori_t = ori_pcs.transpose(0, 2, 1)
    adv_t = adv_pcs.transpose(0, 2, 1)
    tblo, tbla, tbln = pl.pallas_call(
        _limb_body,
        grid=(_B,),
        in_specs=[pl.BlockSpec((1, _N, 3), lambda b: (b, 0, 0))] * 3,
        out_specs=[pl.BlockSpec((1, _N, 9), lambda b: (b, 0, 0))] * 3,
        out_shape=[jax.ShapeDtypeStruct((_B, _N, 9), jnp.bfloat16)] * 3,
        interpret=interpret,
    )(ori_pcs, adv_pcs, ori_normals)
    ko, ka = pl.pallas_call(
        _body,
        grid=(_B, _NBLK),
        in_specs=[
            pl.BlockSpec((1, 3, _N), lambda b, r: (b, 0, 0)),
            pl.BlockSpec((1, 3, _N), lambda b, r: (b, 0, 0)),
            pl.BlockSpec((1, _N, 9), lambda b, r: (b, 0, 0)),
            pl.BlockSpec((1, _N, 9), lambda b, r: (b, 0, 0)),
            pl.BlockSpec((1, _N, 9), lambda b, r: (b, 0, 0)),
            pl.BlockSpec((1, _BR, 3), lambda b, r: (b, r, 0)),
            pl.BlockSpec((1, _BR, 3), lambda b, r: (b, r, 0)),
            pl.BlockSpec((1, _BR, 3), lambda b, r: (b, r, 0)),
        ],
        out_specs=[pl.BlockSpec((1, _BR, 1), lambda b, r: (b, r, 0)),
                   pl.BlockSpec((1, _BR, 1), lambda b, r: (b, r, 0))],
        out_shape=[jax.ShapeDtypeStruct((_B, _N, 1), jnp.float32),
                   jax.ShapeDtypeStruct((_B, _N, 1), jnp.float32)],
        compiler_params=pltpu.CompilerParams(
            dimension_semantics=("arbitrary", "arbitrary")),
        interpret=interpret,
    )(ori_t, adv_t, tblo, tbla, tbln,
      ori_pcs, adv_pcs, ori_normals)
    out = pl.pallas_call(
        _std_body,
        out_shape=jax.ShapeDtypeStruct((1, 1), jnp.float32),
        interpret=interpret,
    )(ko.reshape(_B, _N), ka.reshape(_B, _N))
    return out[0, 0]


def kernel(ori_pcs, adv_pcs, ori_normals):
    return _call(ori_pcs, adv_pcs, ori_normals)
